# baseline (speedup 1.0000x reference)
import jax
import jax.numpy as jnp
from jax import lax
from jax.experimental import pallas as pl
from jax.experimental.pallas import tpu as pltpu
from jax.experimental.pallas import tpu_sc as plsc

BATCH = 16384
D = 64
NC = 2
NS = 16
NW = NC * NS
B_PER_W = BATCH // NW
CHUNK = 128
N_CHUNKS = B_PER_W // CHUNK


def _body(w_team, w_player, w_season, w_down,
          idx_team, idx_player, idx_season, idx_down,
          out,
          iv_t, iv_p, iv_s, iv_d,
          b_t0, b_p0, b_s0, b_d0,
          b_t1, b_p1, b_s1, b_d1,
          o_v0, o_v1,
          sem_g, sem_o):
    wid = lax.axis_index("s") * NC + lax.axis_index("c")
    base = wid * B_PER_W

    bufs = ((b_t0, b_p0, b_s0, b_d0), (b_t1, b_p1, b_s1, b_d1))
    o_vs = (o_v0, o_v1)
    tables = (w_team, w_player, w_season, w_down)
    ivs = (iv_t, iv_p, iv_s, iv_d)

    ih = [pltpu.async_copy(ihbm.at[pl.ds(base, B_PER_W)], iv, sem_g)
          for ihbm, iv in zip((idx_team, idx_player, idx_season, idx_down), ivs)]

    def fire_gathers(c, s):
        sl = pl.ds(c * CHUNK, CHUNK)
        return [pltpu.async_copy(tab.at[iv.at[sl]], buf, sem_g)
                for tab, iv, buf in zip(tables, ivs, bufs[s])]

    gh = []
    sl0 = pl.ds(0, CHUNK)
    for h, tab, iv, buf in zip(ih, tables, ivs, bufs[0]):
        h.wait()
        gh.append(pltpu.async_copy(tab.at[iv.at[sl0]], buf, sem_g))
    oh = [None, None]
    dvecs = [lax.iota(jnp.int32, 16) + jj * 16 for jj in range(D // 16)]
    for c in range(N_CHUNKS):
        cur = c % 2
        for h in gh:
            h.wait()
        if c + 1 < N_CHUNKS:
            gh = fire_gathers(c + 1, 1 - cur)
        if oh[cur] is not None:
            for h in oh[cur]:
                h.wait()
        bt, bp, bs, bd = bufs[cur]
        o_v = o_vs[cur]

        def row(r4, _):
            for u in range(4):
                r = r4 * 4 + u
                col = jnp.full((16,), r, jnp.int32)
                for jj in range(D // 16):
                    sl = pl.ds(jj * 16, 16)
                    acc = (bt[r, sl] + bp[r, sl]) + (bs[r, sl] + bd[r, sl])
                    plsc.store_scatter(o_v, [dvecs[jj], col], acc)
            return 0

        lax.fori_loop(0, CHUNK // 4, row, 0)
        j = wid * N_CHUNKS + c
        oh[cur] = [pltpu.async_copy(
            o_v.at[pl.ds(k * 8, 8), pl.ds(0, CHUNK)], out.at[k, j], sem_o)
            for k in range(D // 8)]
    for hs in oh:
        if hs is not None:
            for h in hs:
                h.wait()


@jax.jit
def _meta_embed(team_ID, player_ids, season_ID, down_ID,
                W_team, W_player, W_season, W_down):
    run = pl.kernel(
        _body,
        out_type=jax.ShapeDtypeStruct((D // 8, BATCH // 128, 8, 128), jnp.float32),
        mesh=plsc.VectorSubcoreMesh(
            core_axis_name="c", subcore_axis_name="s",
            num_cores=NC, num_subcores=NS),
        scratch_types=[
            pltpu.VMEM((B_PER_W,), jnp.int32),
            pltpu.VMEM((B_PER_W,), jnp.int32),
            pltpu.VMEM((B_PER_W,), jnp.int32),
            pltpu.VMEM((B_PER_W,), jnp.int32),
            pltpu.VMEM((CHUNK, D), jnp.float32),
            pltpu.VMEM((CHUNK, D), jnp.float32),
            pltpu.VMEM((CHUNK, D), jnp.float32),
            pltpu.VMEM((CHUNK, D), jnp.float32),
            pltpu.VMEM((CHUNK, D), jnp.float32),
            pltpu.VMEM((CHUNK, D), jnp.float32),
            pltpu.VMEM((CHUNK, D), jnp.float32),
            pltpu.VMEM((CHUNK, D), jnp.float32),
            pltpu.VMEM((D, CHUNK + 5), jnp.float32),
            pltpu.VMEM((D, CHUNK + 5), jnp.float32),
            pltpu.SemaphoreType.DMA,
            pltpu.SemaphoreType.DMA,
        ],
        compiler_params=pltpu.CompilerParams(use_tc_tiling_on_sc=False, needs_layout_passes=False),
    )
    out4 = run(W_team, W_player, W_season, W_down,
               team_ID.astype(jnp.int32), player_ids.astype(jnp.int32),
               season_ID.astype(jnp.int32), down_ID.astype(jnp.int32))
    return jnp.transpose(out4, (1, 3, 0, 2)).reshape(BATCH, D)


def kernel(team_ID, player_ids, season_ID, down_ID,
           W_team, W_player, W_season, W_down):
    return _meta_embed(team_ID, player_ids, season_ID, down_ID,
                       W_team, W_player, W_season, W_down)


# speedup vs baseline: 1.0131x; 1.0131x over previous
import jax
import jax.numpy as jnp
from jax import lax
from jax.experimental import pallas as pl
from jax.experimental.pallas import tpu as pltpu
from jax.experimental.pallas import tpu_sc as plsc

BATCH = 16384
D = 64
NC = 2
NS = 16
NW = NC * NS
B_PER_W = BATCH // NW
CHUNK = 128
N_CHUNKS = B_PER_W // CHUNK


def _body(w_team, w_player, w_season, w_down,
          idx_team, idx_player, idx_season, idx_down,
          out,
          iv_t, iv_p, iv_s, iv_d,
          b_t0, b_p0, b_s0, b_d0,
          b_t1, b_p1, b_s1, b_d1,
          o_v0, o_v1,
          sem_g, sem_o):
    wid = lax.axis_index("s") * NC + lax.axis_index("c")
    base = wid * B_PER_W

    bufs = ((b_t0, b_p0, b_s0, b_d0), (b_t1, b_p1, b_s1, b_d1))
    o_vs = (o_v0, o_v1)
    tables = (w_team, w_player, w_season, w_down)
    ivs = (iv_t, iv_p, iv_s, iv_d)

    ih = [pltpu.async_copy(ihbm.at[pl.ds(base, B_PER_W)], iv, sem_g)
          for ihbm, iv in zip((idx_team, idx_player, idx_season, idx_down), ivs)]

    def fire_gathers(c, s):
        sl = pl.ds(c * CHUNK, CHUNK)
        return [pltpu.async_copy(tab.at[iv.at[sl]], buf, sem_g)
                for tab, iv, buf in zip(tables, ivs, bufs[s])]

    gh = []
    sl0 = pl.ds(0, CHUNK)
    for h, tab, iv, buf in zip(ih, tables, ivs, bufs[0]):
        h.wait()
        gh.append(pltpu.async_copy(tab.at[iv.at[sl0]], buf, sem_g))
    oh = [None, None]
    dvecs = [lax.iota(jnp.int32, 16) + jj * 16 for jj in range(D // 16)]
    for c in range(N_CHUNKS):
        cur = c % 2
        for h in gh:
            h.wait()
        if c + 1 < N_CHUNKS:
            gh = fire_gathers(c + 1, 1 - cur)
        if oh[cur] is not None:
            for h in oh[cur]:
                h.wait()
        bt, bp, bs, bd = bufs[cur]
        o_v = o_vs[cur]

        def row(r2, _):
            for u in range(2):
                r = r2 * 2 + u
                col = jnp.full((16,), r, jnp.int32)
                for jj in range(D // 16):
                    sl = pl.ds(jj * 16, 16)
                    acc = (bt[r, sl] + bp[r, sl]) + (bs[r, sl] + bd[r, sl])
                    plsc.store_scatter(o_v, [dvecs[jj], col], acc)
            return 0

        lax.fori_loop(0, CHUNK // 2, row, 0)
        j = wid * N_CHUNKS + c
        oh[cur] = [pltpu.async_copy(
            o_v.at[pl.ds(k * 8, 8), pl.ds(0, CHUNK)], out.at[k, j], sem_o)
            for k in range(D // 8)]
    for hs in oh:
        if hs is not None:
            for h in hs:
                h.wait()


@jax.jit
def _meta_embed(team_ID, player_ids, season_ID, down_ID,
                W_team, W_player, W_season, W_down):
    run = pl.kernel(
        _body,
        out_type=jax.ShapeDtypeStruct((D // 8, BATCH // 128, 8, 128), jnp.float32),
        mesh=plsc.VectorSubcoreMesh(
            core_axis_name="c", subcore_axis_name="s",
            num_cores=NC, num_subcores=NS),
        scratch_types=[
            pltpu.VMEM((B_PER_W,), jnp.int32),
            pltpu.VMEM((B_PER_W,), jnp.int32),
            pltpu.VMEM((B_PER_W,), jnp.int32),
            pltpu.VMEM((B_PER_W,), jnp.int32),
            pltpu.VMEM((CHUNK, D), jnp.float32),
            pltpu.VMEM((CHUNK, D), jnp.float32),
            pltpu.VMEM((CHUNK, D), jnp.float32),
            pltpu.VMEM((CHUNK, D), jnp.float32),
            pltpu.VMEM((CHUNK, D), jnp.float32),
            pltpu.VMEM((CHUNK, D), jnp.float32),
            pltpu.VMEM((CHUNK, D), jnp.float32),
            pltpu.VMEM((CHUNK, D), jnp.float32),
            pltpu.VMEM((D, CHUNK + 5), jnp.float32),
            pltpu.VMEM((D, CHUNK + 5), jnp.float32),
            pltpu.SemaphoreType.DMA,
            pltpu.SemaphoreType.DMA,
        ],
        compiler_params=pltpu.CompilerParams(use_tc_tiling_on_sc=False, needs_layout_passes=False),
    )
    out4 = run(W_team, W_player, W_season, W_down,
               team_ID.astype(jnp.int32), player_ids.astype(jnp.int32),
               season_ID.astype(jnp.int32), down_ID.astype(jnp.int32))
    return jnp.transpose(out4, (1, 3, 0, 2)).reshape(BATCH, D)


def kernel(team_ID, player_ids, season_ID, down_ID,
           W_team, W_player, W_season, W_down):
    return _meta_embed(team_ID, player_ids, season_ID, down_ID,
                       W_team, W_player, W_season, W_down)
